# baseline (speedup 1.0000x reference)
import functools
import math

import jax
import jax.numpy as jnp
from jax.experimental import pallas as pl
from jax.experimental.pallas import tpu as pltpu
from jax.experimental.pallas import tpu_sc as plsc

_B = 32
_S = 8
_V = 65536
_H = 2048
_HH = 1024
_THRESH = 0.7
_BEAMS = 3
_R = 16
_NB = _R // _S
_SCB = 4
_TCB = _B - _SCB
_TAIL0 = _TCB * _S
_NTAIL = _SCB * _S
_INV_LOG_V = 1.0 / math.log(float(_V))
_INV_SQRT2 = 0.7071067811865476
_L = 16


def _main_body(lg_ref, hs_ref, w1_ref, b1_ref, w2_ref, b2_ref,
               prim_ref, conf_ref, top3_ref, suma_ref):
    i = pl.program_id(0)
    x = lg_ref[...]
    m = jnp.max(x, axis=1, keepdims=True)
    idx = jax.lax.broadcasted_iota(jnp.int32, (_R, _V), 1)
    t = x - m
    amax = jnp.min(jnp.where(t == 0.0, idx, _V), axis=1, keepdims=True)
    e = jnp.exp(t)
    z = jnp.sum(e, axis=1, keepdims=True)
    s1 = jnp.sum(e * t, axis=1, keepdims=True)
    entropy = jnp.log(z) - s1 / z
    norm_ent = entropy * _INV_LOG_V

    h1 = jax.lax.dot_general(hs_ref[...], w1_ref[...],
                             dimension_numbers=(((1,), (1,)), ((), ())),
                             preferred_element_type=jnp.float32)
    h1 = h1 + b1_ref[...]
    g = 0.5 * h1 * (1.0 + jax.lax.erf(h1 * _INV_SQRT2))
    h2 = jnp.sum(g * w2_ref[...], axis=1, keepdims=True)
    lc = jax.nn.sigmoid(h2 + b2_ref[0])

    conf = 0.4 / z + 0.3 * (1.0 - norm_ent) + 0.3 * lc
    prim_ref[...] = amax.reshape(1, _R, 1)
    conf_ref[...] = conf.reshape(1, _R, 1)

    gidx = (jax.lax.broadcasted_iota(jnp.int32, (8, _V // 8), 0) * (_V // 8)
            + jax.lax.broadcasted_iota(jnp.int32, (8, _V // 8), 1))
    tops = []
    for k in range(_NB):
        r = 8 * k + 7
        xr = x[r:r + 1, :].reshape(8, _V // 8)
        v1 = jnp.max(xr)
        i1 = jnp.min(jnp.where(xr == v1, gidx, _V))
        xr = jnp.where(gidx == i1, -jnp.inf, xr)
        v2 = jnp.max(xr)
        i2 = jnp.min(jnp.where(xr == v2, gidx, _V))
        xr = jnp.where(gidx == i2, -jnp.inf, xr)
        v3 = jnp.max(xr)
        i3 = jnp.min(jnp.where(xr == v3, gidx, _V))
        tops += [i1, i2, i3]
    top3_ref[...] = jnp.stack(tops).reshape(1, 1, _NB * _BEAMS)

    @pl.when(i == 0)
    def _init():
        suma_ref[...] = jnp.zeros((1, 1), jnp.float32)

    suma_ref[...] = suma_ref[...] + jnp.sum(conf, axis=0, keepdims=True)


def _sc_stats_body(lg1d, statsf, statsi, row_v, outf_v, outi_v):
    c = jax.lax.axis_index("c")
    s = jax.lax.axis_index("s")
    wid = s * 2 + c
    row = _TAIL0 + wid
    pltpu.sync_copy(lg1d.at[pl.ds(row * _V, _V)], row_v)

    iota16 = jax.lax.iota(jnp.int32, _L)

    def pass1(j, carry):
        m16, i16 = carry
        v = row_v[pl.ds(j * _L, _L)]
        cond = v > m16
        i16 = jnp.where(cond, j * _L + iota16, i16)
        m16 = jnp.where(cond, v, m16)
        return (m16, i16)

    m16, i16 = jax.lax.fori_loop(
        0, _V // _L, pass1,
        (jnp.full((_L,), -jnp.inf, jnp.float32),
         jnp.zeros((_L,), jnp.int32)), unroll=8)
    m = m16[0]
    for j in range(1, _L):
        m = jnp.maximum(m, m16[j])
    cand = jnp.where(m16 == m, i16, _V)
    argm = cand[0]
    for j in range(1, _L):
        argm = jnp.minimum(argm, cand[j])

    def pass2(j, carry):
        z16, s16 = carry
        v = row_v[pl.ds(j * _L, _L)]
        t = v - m
        e = jnp.exp(t)
        return (z16 + e, s16 + e * t)

    z16, s16 = jax.lax.fori_loop(
        0, _V // _L, pass2,
        (jnp.zeros((_L,), jnp.float32),
         jnp.zeros((_L,), jnp.float32)), unroll=8)
    z = z16[0]
    s1 = s16[0]
    for j in range(1, _L):
        z = z + z16[j]
        s1 = s1 + s16[j]

    outf_v[...] = jnp.where(iota16 == 0, m,
                            jnp.where(iota16 == 1, z,
                                      jnp.where(iota16 == 2, s1, 0.0)))
    outi_v[...] = jnp.where(iota16 == 0, argm, 0)
    pltpu.sync_copy(outf_v, statsf.at[pl.ds(wid * _L, _L)])
    pltpu.sync_copy(outi_v, statsi.at[pl.ds(wid * _L, _L)])


def _top3_tail_body(lgr_ref, top3_ref):
    gidx = (jax.lax.broadcasted_iota(jnp.int32, (8, _V // 8), 0) * (_V // 8)
            + jax.lax.broadcasted_iota(jnp.int32, (8, _V // 8), 1))
    xr = lgr_ref[...]
    v1 = jnp.max(xr)
    i1 = jnp.min(jnp.where(xr == v1, gidx, _V))
    xr = jnp.where(gidx == i1, -jnp.inf, xr)
    v2 = jnp.max(xr)
    i2 = jnp.min(jnp.where(xr == v2, gidx, _V))
    xr = jnp.where(gidx == i2, -jnp.inf, xr)
    v3 = jnp.max(xr)
    i3 = jnp.min(jnp.where(xr == v3, gidx, _V))
    top3_ref[...] = jnp.stack([i1, i2, i3]).reshape(1, 1, _BEAMS)


def _combine_body(hs_ref, w1_ref, b1_ref, w2_ref, b2_ref,
                  statsf_ref, statsi_ref, suma_ref, t3m_ref, t3t_ref,
                  prim_ref, conf_ref, mean_ref, alt_ref):
    m = statsf_ref[:, 0:1]
    z = statsf_ref[:, 1:2]
    s1 = statsf_ref[:, 2:3]
    entropy = jnp.log(z) - s1 / z
    norm_ent = entropy * _INV_LOG_V

    h1 = jax.lax.dot_general(hs_ref[...], w1_ref[...],
                             dimension_numbers=(((1,), (1,)), ((), ())),
                             preferred_element_type=jnp.float32)
    h1 = h1 + b1_ref[...]
    g = 0.5 * h1 * (1.0 + jax.lax.erf(h1 * _INV_SQRT2))
    h2 = jnp.sum(g * w2_ref[...], axis=1, keepdims=True)
    lc = jax.nn.sigmoid(h2 + b2_ref[0])

    conf = 0.4 / z + 0.3 * (1.0 - norm_ent) + 0.3 * lc
    prim_ref[...] = statsi_ref[:, 0:1]
    conf_ref[...] = conf
    mean = (suma_ref[...] + jnp.sum(conf, axis=0, keepdims=True)) \
        * (1.0 / (_B * _S))
    mean_ref[...] = mean
    flag = (mean < _THRESH).astype(jnp.int32)
    alt_ref[...] = jnp.concatenate(
        [t3m_ref[...], t3t_ref[...]], axis=0) * flag


def kernel(model, input_ids, logits, hidden_states, W1, b1, W2, b2):
    lg = logits.reshape(_B * _S, _V)
    lg1d = logits.reshape(_B * _S * _V)
    lgr = logits.reshape(_B * _S * 8, _V // 8)
    hs = hidden_states.reshape(_B * _S, _H)
    b1r = b1.reshape(1, _HH)
    w2r = W2.reshape(1, _HH)
    b2r = b2.reshape(1)
    nsteps = _TCB * _S // _R

    sc_mesh = plsc.VectorSubcoreMesh(core_axis_name="c", subcore_axis_name="s")
    sc_stats = functools.partial(
        pl.kernel, mesh=sc_mesh,
        out_type=[
            jax.ShapeDtypeStruct((_NTAIL * _L,), jnp.float32),
            jax.ShapeDtypeStruct((_NTAIL * _L,), jnp.int32),
        ],
        scratch_types=[
            pltpu.VMEM((_V,), jnp.float32),
            pltpu.VMEM((_L,), jnp.float32),
            pltpu.VMEM((_L,), jnp.int32),
        ],
    )(_sc_stats_body)
    statsf, statsi = sc_stats(lg1d)

    prim_main, conf_main, top3_main, suma = pl.pallas_call(
        _main_body,
        grid=(nsteps,),
        in_specs=[
            pl.BlockSpec((_R, _V), lambda i: (i, 0)),
            pl.BlockSpec((_R, _H), lambda i: (i, 0)),
            pl.BlockSpec((_HH, _H), lambda i: (0, 0)),
            pl.BlockSpec((1, _HH), lambda i: (0, 0)),
            pl.BlockSpec((1, _HH), lambda i: (0, 0)),
            pl.BlockSpec(memory_space=pltpu.SMEM),
        ],
        out_specs=[
            pl.BlockSpec((1, _R, 1), lambda i: (i, 0, 0)),
            pl.BlockSpec((1, _R, 1), lambda i: (i, 0, 0)),
            pl.BlockSpec((1, 1, _NB * _BEAMS), lambda i: (i, 0, 0)),
            pl.BlockSpec((1, 1), lambda i: (0, 0)),
        ],
        out_shape=[
            jax.ShapeDtypeStruct((nsteps, _R, 1), jnp.int32),
            jax.ShapeDtypeStruct((nsteps, _R, 1), jnp.float32),
            jax.ShapeDtypeStruct((nsteps, 1, _NB * _BEAMS), jnp.int32),
            jax.ShapeDtypeStruct((1, 1), jnp.float32),
        ],
    )(lg, hs, W1, b1r, w2r, b2r)

    top3_tail = pl.pallas_call(
        _top3_tail_body,
        grid=(_SCB,),
        in_specs=[
            pl.BlockSpec((8, _V // 8), lambda j: (8 * (_TCB + j) + 7, 0)),
        ],
        out_specs=pl.BlockSpec((1, 1, _BEAMS), lambda j: (j, 0, 0)),
        out_shape=jax.ShapeDtypeStruct((_SCB, 1, _BEAMS), jnp.int32),
    )(lgr)

    prim_tail, conf_tail, mean, alternatives = pl.pallas_call(
        _combine_body,
        in_specs=[
            pl.BlockSpec((_NTAIL, _H), lambda: (0, 0)),
            pl.BlockSpec((_HH, _H), lambda: (0, 0)),
            pl.BlockSpec((1, _HH), lambda: (0, 0)),
            pl.BlockSpec((1, _HH), lambda: (0, 0)),
            pl.BlockSpec(memory_space=pltpu.SMEM),
            pl.BlockSpec((_NTAIL, _L), lambda: (0, 0)),
            pl.BlockSpec((_NTAIL, _L), lambda: (0, 0)),
            pl.BlockSpec((1, 1), lambda: (0, 0)),
            pl.BlockSpec((_TCB, _BEAMS), lambda: (0, 0)),
            pl.BlockSpec((_SCB, _BEAMS), lambda: (0, 0)),
        ],
        out_specs=[
            pl.BlockSpec((_NTAIL, 1), lambda: (0, 0)),
            pl.BlockSpec((_NTAIL, 1), lambda: (0, 0)),
            pl.BlockSpec((1, 1), lambda: (0, 0)),
            pl.BlockSpec((_B, _BEAMS), lambda: (0, 0)),
        ],
        out_shape=[
            jax.ShapeDtypeStruct((_NTAIL, 1), jnp.int32),
            jax.ShapeDtypeStruct((_NTAIL, 1), jnp.float32),
            jax.ShapeDtypeStruct((1, 1), jnp.float32),
            jax.ShapeDtypeStruct((_B, _BEAMS), jnp.int32),
        ],
    )(hs[_TAIL0:], W1, b1r, w2r, b2r,
      statsf.reshape(_NTAIL, _L), statsi.reshape(_NTAIL, _L), suma,
      top3_main.reshape(_TCB, _BEAMS), top3_tail.reshape(_SCB, _BEAMS))

    prim = jnp.concatenate(
        [prim_main.reshape(_TCB * _S), prim_tail.reshape(_NTAIL)])
    conf = jnp.concatenate(
        [conf_main.reshape(_TCB * _S), conf_tail.reshape(_NTAIL)])
    return (prim.reshape(_B, _S), conf.reshape(_B, _S),
            mean.reshape(()), alternatives)


# speedup vs baseline: 1.3147x; 1.3147x over previous
import functools
import math

import jax
import jax.numpy as jnp
from jax.experimental import pallas as pl
from jax.experimental.pallas import tpu as pltpu
from jax.experimental.pallas import tpu_sc as plsc

_B = 32
_S = 8
_V = 65536
_H = 2048
_HH = 1024
_THRESH = 0.7
_BEAMS = 3
_R = 16
_NB = _R // _S
_SCB = 4
_TCB = _B - _SCB
_TAIL0 = _TCB * _S
_NTAIL = _SCB * _S
_INV_LOG_V = 1.0 / math.log(float(_V))
_INV_SQRT2 = 0.7071067811865476
_L = 16


def _main_body(lg_ref, hs_ref, w1_ref, b1_ref, w2_ref, b2_ref,
               prim_ref, conf_ref, top3_ref, suma_ref):
    i = pl.program_id(0)
    x = lg_ref[...]
    m = jnp.max(x, axis=1, keepdims=True)
    idx = jax.lax.broadcasted_iota(jnp.int32, (_R, _V), 1)
    t = x - m
    amax = jnp.min(jnp.where(t == 0.0, idx, _V), axis=1, keepdims=True)
    e = jnp.exp(t)
    z = jnp.sum(e, axis=1, keepdims=True)
    s1 = jnp.sum(e * t, axis=1, keepdims=True)
    entropy = jnp.log(z) - s1 / z
    norm_ent = entropy * _INV_LOG_V

    h1 = jax.lax.dot_general(hs_ref[...], w1_ref[...],
                             dimension_numbers=(((1,), (1,)), ((), ())),
                             preferred_element_type=jnp.float32)
    h1 = h1 + b1_ref[...]
    g = 0.5 * h1 * (1.0 + jax.lax.erf(h1 * _INV_SQRT2))
    h2 = jnp.sum(g * w2_ref[...], axis=1, keepdims=True)
    lc = jax.nn.sigmoid(h2 + b2_ref[0])

    conf = 0.4 / z + 0.3 * (1.0 - norm_ent) + 0.3 * lc
    prim_ref[...] = amax.reshape(1, _R, 1)
    conf_ref[...] = conf.reshape(1, _R, 1)

    gidx = (jax.lax.broadcasted_iota(jnp.int32, (8, _V // 8), 0) * (_V // 8)
            + jax.lax.broadcasted_iota(jnp.int32, (8, _V // 8), 1))
    tops = []
    for k in range(_NB):
        r = 8 * k + 7
        xr = x[r:r + 1, :].reshape(8, _V // 8)
        v1 = jnp.max(xr)
        i1 = jnp.min(jnp.where(xr == v1, gidx, _V))
        xr = jnp.where(gidx == i1, -jnp.inf, xr)
        v2 = jnp.max(xr)
        i2 = jnp.min(jnp.where(xr == v2, gidx, _V))
        xr = jnp.where(gidx == i2, -jnp.inf, xr)
        v3 = jnp.max(xr)
        i3 = jnp.min(jnp.where(xr == v3, gidx, _V))
        tops += [i1, i2, i3]
    top3_ref[...] = jnp.stack(tops).reshape(1, 1, _NB * _BEAMS)

    @pl.when(i == 0)
    def _init():
        suma_ref[...] = jnp.zeros((1, 1), jnp.float32)

    suma_ref[...] = suma_ref[...] + jnp.sum(conf, axis=0, keepdims=True)


def _sc_stats_body(lg2d, statsf, statsi, row_v, outf_v, outi_v):
    c = jax.lax.axis_index("c")
    s = jax.lax.axis_index("s")
    wid = s * 2 + c
    row = _TAIL0 + wid
    pltpu.sync_copy(lg2d.at[row], row_v)

    iota16 = jax.lax.iota(jnp.int32, _L)

    def pass1(j, carry):
        m16, i16 = carry
        v = row_v[pl.ds(j * _L, _L)]
        cond = v > m16
        i16 = jnp.where(cond, j * _L + iota16, i16)
        m16 = jnp.where(cond, v, m16)
        return (m16, i16)

    m16, i16 = jax.lax.fori_loop(
        0, _V // _L, pass1,
        (jnp.full((_L,), -jnp.inf, jnp.float32),
         jnp.zeros((_L,), jnp.int32)), unroll=8)
    m = m16[0]
    for j in range(1, _L):
        m = jnp.maximum(m, m16[j])
    cand = jnp.where(m16 == m, i16, _V)
    argm = cand[0]
    for j in range(1, _L):
        argm = jnp.minimum(argm, cand[j])

    def pass2(j, carry):
        z16, s16 = carry
        v = row_v[pl.ds(j * _L, _L)]
        t = v - m
        e = jnp.exp(t)
        return (z16 + e, s16 + e * t)

    z16, s16 = jax.lax.fori_loop(
        0, _V // _L, pass2,
        (jnp.zeros((_L,), jnp.float32),
         jnp.zeros((_L,), jnp.float32)), unroll=8)
    z = z16[0]
    s1 = s16[0]
    for j in range(1, _L):
        z = z + z16[j]
        s1 = s1 + s16[j]

    outf_v[...] = jnp.where(iota16 == 0, m,
                            jnp.where(iota16 == 1, z,
                                      jnp.where(iota16 == 2, s1, 0.0)))
    outi_v[...] = jnp.where(iota16 == 0, argm, 0)
    pltpu.sync_copy(outf_v, statsf.at[pl.ds(wid * _L, _L)])
    pltpu.sync_copy(outi_v, statsi.at[pl.ds(wid * _L, _L)])


def _top3_tail_body(lgr_ref, top3_ref):
    gidx = (jax.lax.broadcasted_iota(jnp.int32, (8, _V // 8), 0) * (_V // 8)
            + jax.lax.broadcasted_iota(jnp.int32, (8, _V // 8), 1))
    xr = lgr_ref[...]
    v1 = jnp.max(xr)
    i1 = jnp.min(jnp.where(xr == v1, gidx, _V))
    xr = jnp.where(gidx == i1, -jnp.inf, xr)
    v2 = jnp.max(xr)
    i2 = jnp.min(jnp.where(xr == v2, gidx, _V))
    xr = jnp.where(gidx == i2, -jnp.inf, xr)
    v3 = jnp.max(xr)
    i3 = jnp.min(jnp.where(xr == v3, gidx, _V))
    top3_ref[...] = jnp.stack([i1, i2, i3]).reshape(1, 1, _BEAMS)


def _combine_body(hs_ref, w1_ref, b1_ref, w2_ref, b2_ref,
                  statsf_ref, statsi_ref, suma_ref, t3m_ref, t3t_ref,
                  prim_ref, conf_ref, mean_ref, alt_ref):
    m = statsf_ref[:, 0:1]
    z = statsf_ref[:, 1:2]
    s1 = statsf_ref[:, 2:3]
    entropy = jnp.log(z) - s1 / z
    norm_ent = entropy * _INV_LOG_V

    h1 = jax.lax.dot_general(hs_ref[...], w1_ref[...],
                             dimension_numbers=(((1,), (1,)), ((), ())),
                             preferred_element_type=jnp.float32)
    h1 = h1 + b1_ref[...]
    g = 0.5 * h1 * (1.0 + jax.lax.erf(h1 * _INV_SQRT2))
    h2 = jnp.sum(g * w2_ref[...], axis=1, keepdims=True)
    lc = jax.nn.sigmoid(h2 + b2_ref[0])

    conf = 0.4 / z + 0.3 * (1.0 - norm_ent) + 0.3 * lc
    prim_ref[...] = statsi_ref[:, 0:1]
    conf_ref[...] = conf
    mean = (suma_ref[...] + jnp.sum(conf, axis=0, keepdims=True)) \
        * (1.0 / (_B * _S))
    mean_ref[...] = mean
    flag = (mean < _THRESH).astype(jnp.int32)
    alt_ref[...] = jnp.concatenate(
        [t3m_ref[...], t3t_ref[...]], axis=0) * flag


def kernel(model, input_ids, logits, hidden_states, W1, b1, W2, b2):
    lg = logits.reshape(_B * _S, _V)
    lgr = logits.reshape(_B * _S * 8, _V // 8)
    hs = hidden_states.reshape(_B * _S, _H)
    b1r = b1.reshape(1, _HH)
    w2r = W2.reshape(1, _HH)
    b2r = b2.reshape(1)
    nsteps = _TCB * _S // _R

    sc_mesh = plsc.VectorSubcoreMesh(core_axis_name="c", subcore_axis_name="s")
    sc_stats = functools.partial(
        pl.kernel, mesh=sc_mesh,
        out_type=[
            jax.ShapeDtypeStruct((_NTAIL * _L,), jnp.float32),
            jax.ShapeDtypeStruct((_NTAIL * _L,), jnp.int32),
        ],
        scratch_types=[
            pltpu.VMEM((_V,), jnp.float32),
            pltpu.VMEM((_L,), jnp.float32),
            pltpu.VMEM((_L,), jnp.int32),
        ],
    )(_sc_stats_body)
    statsf, statsi = sc_stats(lg)

    prim_main, conf_main, top3_main, suma = pl.pallas_call(
        _main_body,
        grid=(nsteps,),
        in_specs=[
            pl.BlockSpec((_R, _V), lambda i: (i, 0)),
            pl.BlockSpec((_R, _H), lambda i: (i, 0)),
            pl.BlockSpec((_HH, _H), lambda i: (0, 0)),
            pl.BlockSpec((1, _HH), lambda i: (0, 0)),
            pl.BlockSpec((1, _HH), lambda i: (0, 0)),
            pl.BlockSpec(memory_space=pltpu.SMEM),
        ],
        out_specs=[
            pl.BlockSpec((1, _R, 1), lambda i: (i, 0, 0)),
            pl.BlockSpec((1, _R, 1), lambda i: (i, 0, 0)),
            pl.BlockSpec((1, 1, _NB * _BEAMS), lambda i: (i, 0, 0)),
            pl.BlockSpec((1, 1), lambda i: (0, 0)),
        ],
        out_shape=[
            jax.ShapeDtypeStruct((nsteps, _R, 1), jnp.int32),
            jax.ShapeDtypeStruct((nsteps, _R, 1), jnp.float32),
            jax.ShapeDtypeStruct((nsteps, 1, _NB * _BEAMS), jnp.int32),
            jax.ShapeDtypeStruct((1, 1), jnp.float32),
        ],
    )(lg, hs, W1, b1r, w2r, b2r)

    top3_tail = pl.pallas_call(
        _top3_tail_body,
        grid=(_SCB,),
        in_specs=[
            pl.BlockSpec((8, _V // 8), lambda j: (8 * (_TCB + j) + 7, 0)),
        ],
        out_specs=pl.BlockSpec((1, 1, _BEAMS), lambda j: (j, 0, 0)),
        out_shape=jax.ShapeDtypeStruct((_SCB, 1, _BEAMS), jnp.int32),
    )(lgr)

    prim_tail, conf_tail, mean, alternatives = pl.pallas_call(
        _combine_body,
        in_specs=[
            pl.BlockSpec((_NTAIL, _H), lambda: (0, 0)),
            pl.BlockSpec((_HH, _H), lambda: (0, 0)),
            pl.BlockSpec((1, _HH), lambda: (0, 0)),
            pl.BlockSpec((1, _HH), lambda: (0, 0)),
            pl.BlockSpec(memory_space=pltpu.SMEM),
            pl.BlockSpec((_NTAIL, _L), lambda: (0, 0)),
            pl.BlockSpec((_NTAIL, _L), lambda: (0, 0)),
            pl.BlockSpec((1, 1), lambda: (0, 0)),
            pl.BlockSpec((_TCB, _BEAMS), lambda: (0, 0)),
            pl.BlockSpec((_SCB, _BEAMS), lambda: (0, 0)),
        ],
        out_specs=[
            pl.BlockSpec((_NTAIL, 1), lambda: (0, 0)),
            pl.BlockSpec((_NTAIL, 1), lambda: (0, 0)),
            pl.BlockSpec((1, 1), lambda: (0, 0)),
            pl.BlockSpec((_B, _BEAMS), lambda: (0, 0)),
        ],
        out_shape=[
            jax.ShapeDtypeStruct((_NTAIL, 1), jnp.int32),
            jax.ShapeDtypeStruct((_NTAIL, 1), jnp.float32),
            jax.ShapeDtypeStruct((1, 1), jnp.float32),
            jax.ShapeDtypeStruct((_B, _BEAMS), jnp.int32),
        ],
    )(hs[_TAIL0:], W1, b1r, w2r, b2r,
      statsf.reshape(_NTAIL, _L), statsi.reshape(_NTAIL, _L), suma,
      top3_main.reshape(_TCB, _BEAMS), top3_tail.reshape(_SCB, _BEAMS))

    prim = jnp.concatenate(
        [prim_main.reshape(_TCB * _S), prim_tail.reshape(_NTAIL)])
    conf = jnp.concatenate(
        [conf_main.reshape(_TCB * _S), conf_tail.reshape(_NTAIL)])
    return (prim.reshape(_B, _S), conf.reshape(_B, _S),
            mean.reshape(()), alternatives)


# speedup vs baseline: 2.7164x; 2.0662x over previous
import math

import jax
import jax.numpy as jnp
from jax.experimental import pallas as pl
from jax.experimental.pallas import tpu as pltpu

_B = 32
_S = 8
_V = 65536
_H = 2048
_HH = 1024
_THRESH = 0.7
_BEAMS = 3
_R = 16
_NB = _R // _S
_INV_LOG_V = 1.0 / math.log(float(_V))
_INV_SQRT2 = 0.7071067811865476


def _main_body(lg_ref, hs_ref, w1_ref, b1_ref, w2_ref, b2_ref,
               prim_ref, conf_ref, top3_ref, mean_ref):
    i = pl.program_id(0)
    x = lg_ref[...]
    m = jnp.max(x, axis=1, keepdims=True)
    idx = jax.lax.broadcasted_iota(jnp.int32, (_R, _V), 1)
    t = x - m
    amax = jnp.min(jnp.where(t == 0.0, idx, _V), axis=1, keepdims=True)
    e = jnp.exp(t)
    z = jnp.sum(e, axis=1, keepdims=True)
    s1 = jnp.sum(e * t, axis=1, keepdims=True)
    entropy = jnp.log(z) - s1 / z
    norm_ent = entropy * _INV_LOG_V

    h1 = jax.lax.dot_general(hs_ref[...], w1_ref[...],
                             dimension_numbers=(((1,), (1,)), ((), ())),
                             preferred_element_type=jnp.float32)
    h1 = h1 + b1_ref[...]
    g = 0.5 * h1 * (1.0 + jax.lax.erf(h1 * _INV_SQRT2))
    h2 = jnp.sum(g * w2_ref[...], axis=1, keepdims=True)
    lc = jax.nn.sigmoid(h2 + b2_ref[0])

    conf = 0.4 / z + 0.3 * (1.0 - norm_ent) + 0.3 * lc
    prim_ref[...] = amax.reshape(1, _R, 1)
    conf_ref[...] = conf.reshape(1, _R, 1)

    gidx = (jax.lax.broadcasted_iota(jnp.int32, (8, _V // 8), 0) * (_V // 8)
            + jax.lax.broadcasted_iota(jnp.int32, (8, _V // 8), 1))
    tops = []
    for k in range(_NB):
        r = 8 * k + 7
        xr = x[r:r + 1, :].reshape(8, _V // 8)
        v1 = jnp.max(xr)
        i1 = jnp.min(jnp.where(xr == v1, gidx, _V))
        xr = jnp.where(gidx == i1, -jnp.inf, xr)
        v2 = jnp.max(xr)
        i2 = jnp.min(jnp.where(xr == v2, gidx, _V))
        xr = jnp.where(gidx == i2, -jnp.inf, xr)
        v3 = jnp.max(xr)
        i3 = jnp.min(jnp.where(xr == v3, gidx, _V))
        tops += [i1, i2, i3]
    top3_ref[...] = jnp.stack(tops).reshape(1, 1, _NB * _BEAMS)

    @pl.when(i == 0)
    def _init():
        mean_ref[...] = jnp.zeros((1, 1), jnp.float32)

    mean_ref[...] = mean_ref[...] + jnp.sum(conf, axis=0, keepdims=True)

    @pl.when(i == pl.num_programs(0) - 1)
    def _fin():
        mean_ref[...] = mean_ref[...] * (1.0 / (_B * _S))


def _flag_body(top3_ref, mean_ref, alt_ref):
    flag = (mean_ref[...] < _THRESH).astype(jnp.int32)
    alt_ref[...] = top3_ref[...] * flag


def kernel(model, input_ids, logits, hidden_states, W1, b1, W2, b2):
    lg = logits.reshape(_B * _S, _V)
    hs = hidden_states.reshape(_B * _S, _H)
    b1r = b1.reshape(1, _HH)
    w2r = W2.reshape(1, _HH)
    b2r = b2.reshape(1)
    nsteps = _B * _S // _R

    prim, conf, top3, mean = pl.pallas_call(
        _main_body,
        grid=(nsteps,),
        in_specs=[
            pl.BlockSpec((_R, _V), lambda i: (i, 0)),
            pl.BlockSpec((_R, _H), lambda i: (i, 0)),
            pl.BlockSpec((_HH, _H), lambda i: (0, 0)),
            pl.BlockSpec((1, _HH), lambda i: (0, 0)),
            pl.BlockSpec((1, _HH), lambda i: (0, 0)),
            pl.BlockSpec(memory_space=pltpu.SMEM),
        ],
        out_specs=[
            pl.BlockSpec((1, _R, 1), lambda i: (i, 0, 0)),
            pl.BlockSpec((1, _R, 1), lambda i: (i, 0, 0)),
            pl.BlockSpec((1, 1, _NB * _BEAMS), lambda i: (i, 0, 0)),
            pl.BlockSpec((1, 1), lambda i: (0, 0)),
        ],
        out_shape=[
            jax.ShapeDtypeStruct((nsteps, _R, 1), jnp.int32),
            jax.ShapeDtypeStruct((nsteps, _R, 1), jnp.float32),
            jax.ShapeDtypeStruct((nsteps, 1, _NB * _BEAMS), jnp.int32),
            jax.ShapeDtypeStruct((1, 1), jnp.float32),
        ],
    )(lg, hs, W1, b1r, w2r, b2r)

    alternatives = pl.pallas_call(
        _flag_body,
        in_specs=[
            pl.BlockSpec((_B, _BEAMS), lambda: (0, 0)),
            pl.BlockSpec((1, 1), lambda: (0, 0)),
        ],
        out_specs=pl.BlockSpec((_B, _BEAMS), lambda: (0, 0)),
        out_shape=jax.ShapeDtypeStruct((_B, _BEAMS), jnp.int32),
    )(top3.reshape(_B, _BEAMS), mean)

    return (prim.reshape(_B, _S), conf.reshape(_B, _S),
            mean.reshape(()), alternatives)
